# Initial kernel scaffold; baseline (speedup 1.0000x reference)
import jax
import jax.numpy as jnp
from jax.experimental import pallas as pl


def kernel(times, spaces, time_emb, space_emb):
    raise NotImplementedError("write your pallas kernel here")


# speedup vs baseline: 5.5717x; 5.5717x over previous
import functools

import jax
import jax.numpy as jnp
from jax import lax
from jax.experimental import pallas as pl
from jax.experimental.pallas import tpu as pltpu
from jax.experimental.pallas import tpu_sc as plsc

DIM = 64
NC = 2
NS = 16
NW = NC * NS
CHUNK = 128


@functools.lru_cache(maxsize=None)
def _make_lookup(n_chunks):
  mesh = plsc.VectorSubcoreMesh(core_axis_name="c", subcore_axis_name="s")

  @functools.partial(
      pl.kernel,
      mesh=mesh,
      compiler_params=pltpu.CompilerParams(use_tc_tiling_on_sc=False),
      out_type=jax.ShapeDtypeStruct((NW, n_chunks, CHUNK, DIM), jnp.float32),
      scratch_types=[
          pltpu.VMEM((n_chunks, CHUNK), jnp.int32),
          pltpu.VMEM((n_chunks, CHUNK), jnp.int32),
          pltpu.VMEM((CHUNK, DIM), jnp.float32),
          pltpu.VMEM((CHUNK, DIM), jnp.float32),
          pltpu.SemaphoreType.DMA,
      ],
  )
  def lookup(t_tab, s_tab, t_idx, s_idx, out, tiv, siv, bt, bs, sem):
    wid = lax.axis_index("s") * NC + lax.axis_index("c")
    pltpu.sync_copy(t_idx.at[wid], tiv)
    pltpu.sync_copy(s_idx.at[wid], siv)

    def chunk_body(c, carry):
      cp_t = pltpu.async_copy(t_tab.at[tiv.at[c]], bt, sem)
      cp_s = pltpu.async_copy(s_tab.at[siv.at[c]], bs, sem)
      cp_t.wait()
      cp_s.wait()

      def row_body(i, carry2):
        for j in range(DIM // 16):
          sl = pl.ds(j * 16, 16)
          bt[i, sl] = bt[i, sl] + bs[i, sl]
        return carry2

      lax.fori_loop(0, CHUNK, row_body, 0)
      pltpu.sync_copy(bt, out.at[wid, c])
      return carry

    lax.fori_loop(0, n_chunks, chunk_body, 0)

  return lookup


def kernel(times, spaces, time_emb, space_emb):
  B, L = times.shape
  n = B * L
  assert n % (NW * CHUNK) == 0
  n_chunks = n // (NW * CHUNK)
  t_idx = times.reshape(NW, n_chunks, CHUNK).astype(jnp.int32)
  s_idx = spaces.reshape(NW, n_chunks, CHUNK).astype(jnp.int32)
  out = _make_lookup(n_chunks)(time_emb, space_emb, t_idx, s_idx)
  return out.reshape(B, L, DIM)
